# baseline (speedup 1.0000x reference)
import functools

import jax
import jax.numpy as jnp
from jax.experimental import pallas as pl

N_ROW = 4096
IN_DIM = 256
EMB = 64
BLK = 4096


def _gnn_block(x_ref, emb_ref, wl1t_ref, bl1_ref, wr1t_ref, wl2t_ref,
               bl2_ref, wr2t_ref, out_ref):
    xf = x_ref[...].astype(jnp.float32)
    m1 = jnp.dot(xf, emb_ref[...], preferred_element_type=jnp.float32)
    rsum = jnp.sum(xf, axis=1, keepdims=True)
    rs = jnp.maximum(rsum, 1.0)
    bl1 = bl1_ref[...]
    mean1 = m1 / rs
    hs1 = jnp.dot(mean1, wl1t_ref[...], preferred_element_type=jnp.float32) + bl1
    m2 = jnp.dot(m1, wr1t_ref[...], preferred_element_type=jnp.float32) + rsum * bl1
    mean2 = m2 / rs
    out_ref[...] = (
        jnp.dot(mean2, wl2t_ref[...], preferred_element_type=jnp.float32)
        + bl2_ref[...]
        + jnp.dot(hs1, wr2t_ref[...], preferred_element_type=jnp.float32)
    )


@jax.jit
def kernel(x, emb, W_l1, b_l1, W_r1, W_l2, b_l2, W_r2):
    n_row, n_col = x.shape
    e = emb.shape[1]
    grid = (n_row // BLK,)
    small = pl.BlockSpec((IN_DIM, EMB), lambda i: (0, 0))
    wspec = pl.BlockSpec((EMB, EMB), lambda i: (0, 0))
    bspec = pl.BlockSpec((1, EMB), lambda i: (0, 0))
    out = pl.pallas_call(
        _gnn_block,
        grid=grid,
        in_specs=[
            pl.BlockSpec((BLK, IN_DIM), lambda i: (i, 0)),
            small,
            wspec, bspec, wspec, wspec, bspec, wspec,
        ],
        out_specs=pl.BlockSpec((BLK, EMB), lambda i: (i, 0)),
        out_shape=jax.ShapeDtypeStruct((n_row, e), jnp.float32),
    )(
        x,
        emb,
        W_l1.T, b_l1.reshape(1, e), W_r1.T,
        W_l2.T, b_l2.reshape(1, e), W_r2.T,
    )
    return out


# speedup vs baseline: 1.0309x; 1.0309x over previous
import functools

import jax
import jax.numpy as jnp
from jax.experimental import pallas as pl

N_ROW = 4096
IN_DIM = 256
EMB = 64
BLK = 2048


def _gnn_block(x_ref, emb_ref, wl1t_ref, bl1_ref, wr1t_ref, wl2t_ref,
               bl2_ref, wr2t_ref, out_ref):
    xf = x_ref[...].astype(jnp.float32)
    m1 = jnp.dot(xf, emb_ref[...], preferred_element_type=jnp.float32)
    rsum = jnp.sum(xf, axis=1, keepdims=True)
    rs = jnp.maximum(rsum, 1.0)
    bl1 = bl1_ref[...]
    mean1 = m1 / rs
    hs1 = jnp.dot(mean1, wl1t_ref[...], preferred_element_type=jnp.float32) + bl1
    m2 = jnp.dot(m1, wr1t_ref[...], preferred_element_type=jnp.float32) + rsum * bl1
    mean2 = m2 / rs
    out_ref[...] = (
        jnp.dot(mean2, wl2t_ref[...], preferred_element_type=jnp.float32)
        + bl2_ref[...]
        + jnp.dot(hs1, wr2t_ref[...], preferred_element_type=jnp.float32)
    )


@jax.jit
def kernel(x, emb, W_l1, b_l1, W_r1, W_l2, b_l2, W_r2):
    n_row, n_col = x.shape
    e = emb.shape[1]
    grid = (n_row // BLK,)
    small = pl.BlockSpec((IN_DIM, EMB), lambda i: (0, 0))
    wspec = pl.BlockSpec((EMB, EMB), lambda i: (0, 0))
    bspec = pl.BlockSpec((1, EMB), lambda i: (0, 0))
    out = pl.pallas_call(
        _gnn_block,
        grid=grid,
        in_specs=[
            pl.BlockSpec((BLK, IN_DIM), lambda i: (i, 0)),
            small,
            wspec, bspec, wspec, wspec, bspec, wspec,
        ],
        out_specs=pl.BlockSpec((BLK, EMB), lambda i: (i, 0)),
        out_shape=jax.ShapeDtypeStruct((n_row, e), jnp.float32),
    )(
        x,
        emb,
        W_l1.T, b_l1.reshape(1, e), W_r1.T,
        W_l2.T, b_l2.reshape(1, e), W_r2.T,
    )
    return out
